# baseline (speedup 1.0000x reference)
import functools

import jax
import jax.numpy as jnp
from jax import lax
from jax.experimental import pallas as pl
from jax.experimental.pallas import tpu as pltpu
from jax.experimental.pallas import tpu_sc as plsc

_DIM = 4096
_NE = 64
_NTOK = 32768
_TB = 1024

_NW = 32
_RW = _NTOK // _NW
_RG = 16


def _softmax_body(x_ref, w_ref, b_ref, probs_ref):
    x = x_ref[...]
    w = w_ref[...]
    logits = lax.dot_general(x, w, (((1,), (1,)), ((), ())))
    logits = logits + b_ref[...]
    m = jnp.max(logits, axis=1, keepdims=True)
    e = jnp.exp(logits - m)
    s = jnp.sum(e, axis=1, keepdims=True)
    probs_ref[...] = e / s


def _tc_softmax(x, W, b):
    ntok = x.shape[0]
    return pl.pallas_call(
        _softmax_body,
        grid=(ntok // _TB,),
        in_specs=[
            pl.BlockSpec((_TB, _DIM), lambda i: (i, 0)),
            pl.BlockSpec((_NE, _DIM), lambda i: (0, 0)),
            pl.BlockSpec((1, _NE), lambda i: (0, 0)),
        ],
        out_specs=pl.BlockSpec((_TB, _NE), lambda i: (i, 0)),
        out_shape=jax.ShapeDtypeStruct((ntok, _NE), jnp.float32),
    )(x, W, b.reshape(1, _NE))


@functools.partial(
    pl.kernel,
    mesh=plsc.VectorSubcoreMesh(core_axis_name="c", subcore_axis_name="s"),
    compiler_params=pltpu.CompilerParams(needs_layout_passes=False),
    out_type=[
        jax.ShapeDtypeStruct((_NTOK,), jnp.float32),
        jax.ShapeDtypeStruct((_NTOK,), jnp.int32),
    ],
    scratch_types=[
        pltpu.VMEM((_RW * _NE,), jnp.float32),
        pltpu.VMEM((_RW,), jnp.float32),
        pltpu.VMEM((_RW,), jnp.int32),
    ],
)
def _sc_top1(probs_hbm, w_hbm, i_hbm, p_v, w_v, i_v):
    wid = lax.axis_index("s") * 2 + lax.axis_index("c")
    base = wid * _RW
    pltpu.sync_copy(probs_hbm.at[pl.ds(base * _NE, _RW * _NE)], p_v)

    lane_off = lax.iota(jnp.int32, _RG) * _NE
    _NCH = 4

    def group(g, carry):
        r0 = g * _RG
        flat0 = r0 * _NE
        best = [jnp.full((_RG,), -1.0, jnp.float32) for _ in range(_NCH)]
        best_i = [jnp.zeros((_RG,), jnp.int32) for _ in range(_NCH)]
        for e in range(_NE):
            c = e % _NCH
            v = plsc.load_gather(p_v, [lane_off + (flat0 + e)])
            upd = v > best[c]
            best[c] = jnp.maximum(v, best[c])
            best_i[c] = jnp.where(upd, e, best_i[c])
        m, mi = best[0], best_i[0]
        for c in range(1, _NCH):
            take = (best[c] > m) | ((best[c] == m) & (best_i[c] < mi))
            m = jnp.where(take, best[c], m)
            mi = jnp.where(take, best_i[c], mi)
        w_v[pl.ds(r0, _RG)] = m
        i_v[pl.ds(r0, _RG)] = mi
        return carry

    lax.fori_loop(0, _RW // _RG, group, 0)
    pltpu.sync_copy(w_v, w_hbm.at[pl.ds(base, _RW)])
    pltpu.sync_copy(i_v, i_hbm.at[pl.ds(base, _RW)])


def kernel(x, W, b):
    probs = _tc_softmax(x, W, b)
    wts, idx = _sc_top1(probs.reshape(-1))
    return (wts.reshape(-1, 1), idx.reshape(-1, 1), probs)


# speedup vs baseline: 1.0776x; 1.0776x over previous
import functools

import jax
import jax.numpy as jnp
from jax import lax
from jax.experimental import pallas as pl
from jax.experimental.pallas import tpu as pltpu
from jax.experimental.pallas import tpu_sc as plsc

_DIM = 4096
_NE = 64
_NTOK = 32768
_TB = 1024

_NW = 32
_RW = _NTOK // _NW
_RG = 16


def _softmax_body(x_ref, w_ref, b_ref, probs_ref):
    x = x_ref[...]
    w = w_ref[...]
    logits = lax.dot_general(x, w, (((1,), (1,)), ((), ())))
    logits = logits + b_ref[...]
    m = jnp.max(logits, axis=1, keepdims=True)
    e = jnp.exp(logits - m)
    s = jnp.sum(e, axis=1, keepdims=True)
    probs_ref[...] = e / s


def _tc_softmax(x, W, b):
    ntok = x.shape[0]
    return pl.pallas_call(
        _softmax_body,
        grid=(ntok // _TB,),
        in_specs=[
            pl.BlockSpec((_TB, _DIM), lambda i: (i, 0)),
            pl.BlockSpec((_NE, _DIM), lambda i: (0, 0)),
            pl.BlockSpec((1, _NE), lambda i: (0, 0)),
        ],
        out_specs=pl.BlockSpec((_TB, _NE), lambda i: (i, 0)),
        out_shape=jax.ShapeDtypeStruct((ntok, _NE), jnp.float32),
    )(x, W, b.reshape(1, _NE))


@functools.partial(
    pl.kernel,
    mesh=plsc.VectorSubcoreMesh(core_axis_name="c", subcore_axis_name="s"),
    compiler_params=pltpu.CompilerParams(needs_layout_passes=False),
    out_type=[
        jax.ShapeDtypeStruct((_NTOK,), jnp.float32),
        jax.ShapeDtypeStruct((_NTOK,), jnp.int32),
    ],
    scratch_types=[
        pltpu.VMEM((_RW * _NE,), jnp.float32),
        pltpu.VMEM((_RW,), jnp.float32),
        pltpu.VMEM((_RW,), jnp.int32),
    ],
)
def _sc_top1(probs_hbm, w_hbm, i_hbm, p_v, w_v, i_v):
    wid = lax.axis_index("s") * 2 + lax.axis_index("c")
    base = wid * _RW
    pltpu.sync_copy(probs_hbm.at[pl.ds(base * _NE, _RW * _NE)], p_v)

    lanes = lax.iota(jnp.int32, _RG)
    diag_off = lanes * _NE + lanes
    _NCH = 4

    def group(g, carry):
        flat0 = g * (_RG * _NE)
        base_idx = diag_off + flat0
        best = [jnp.full((_RG,), -1.0, jnp.float32) for _ in range(_NCH)]
        best_i = [jnp.zeros((_RG,), jnp.int32) for _ in range(_NCH)]
        for d in range(_NE):
            c = d % _NCH
            e_vec = jnp.bitwise_and(lanes + d, _NE - 1)
            idx = base_idx + jnp.where(lanes + d >= _NE, d - _NE, d)
            v = plsc.load_gather(p_v, [idx])
            upd = v > best[c]
            best[c] = jnp.maximum(v, best[c])
            best_i[c] = jnp.where(upd, e_vec, best_i[c])
        m, mi = best[0], best_i[0]
        for c in range(1, _NCH):
            take = (best[c] > m) | ((best[c] == m) & (best_i[c] < mi))
            m = jnp.where(take, best[c], m)
            mi = jnp.where(take, best_i[c], mi)
        r0 = g * _RG
        w_v[pl.ds(r0, _RG)] = m
        i_v[pl.ds(r0, _RG)] = mi
        return carry

    lax.fori_loop(0, _RW // _RG, group, 0)
    pltpu.sync_copy(w_v, w_hbm.at[pl.ds(base, _RW)])
    pltpu.sync_copy(i_v, i_hbm.at[pl.ds(base, _RW)])


def kernel(x, W, b):
    probs = _tc_softmax(x, W, b)
    wts, idx = _sc_top1(probs.reshape(-1))
    return (wts.reshape(-1, 1), idx.reshape(-1, 1), probs)


# speedup vs baseline: 1.0898x; 1.0113x over previous
import functools

import jax
import jax.numpy as jnp
from jax import lax
from jax.experimental import pallas as pl
from jax.experimental.pallas import tpu as pltpu
from jax.experimental.pallas import tpu_sc as plsc

_DIM = 4096
_NE = 64
_NTOK = 32768
_TB = 1024

_NW = 32
_RW = _NTOK // _NW
_RG = 16


def _softmax_body(x_ref, w_ref, b_ref, probs_ref):
    x = x_ref[...]
    w = w_ref[...]
    logits = lax.dot_general(x, w, (((1,), (1,)), ((), ())))
    logits = logits + b_ref[...]
    m = jnp.max(logits, axis=1, keepdims=True)
    e = jnp.exp(logits - m)
    s = jnp.sum(e, axis=1, keepdims=True)
    probs_ref[...] = e / s


def _tc_softmax(x, W, b):
    ntok = x.shape[0]
    return pl.pallas_call(
        _softmax_body,
        grid=(ntok // _TB,),
        in_specs=[
            pl.BlockSpec((_TB, _DIM), lambda i: (i, 0)),
            pl.BlockSpec((_NE, _DIM), lambda i: (0, 0)),
            pl.BlockSpec((1, _NE), lambda i: (0, 0)),
        ],
        out_specs=pl.BlockSpec((_TB, _NE), lambda i: (i, 0)),
        out_shape=jax.ShapeDtypeStruct((ntok, _NE), jnp.float32),
    )(x, W, b.reshape(1, _NE))


@functools.partial(
    pl.kernel,
    mesh=plsc.VectorSubcoreMesh(core_axis_name="c", subcore_axis_name="s"),
    compiler_params=pltpu.CompilerParams(needs_layout_passes=False),
    out_type=[
        jax.ShapeDtypeStruct((_NTOK,), jnp.float32),
        jax.ShapeDtypeStruct((_NTOK,), jnp.int32),
    ],
    scratch_types=[
        pltpu.VMEM((_RW * _NE,), jnp.float32),
        pltpu.VMEM((_RW,), jnp.float32),
        pltpu.VMEM((_RW,), jnp.int32),
    ],
)
def _sc_top1(probs_hbm, w_hbm, i_hbm, p_v, w_v, i_v):
    wid = lax.axis_index("s") * 2 + lax.axis_index("c")
    base = wid * _RW
    pltpu.sync_copy(probs_hbm.at[pl.ds(base * _NE, _RW * _NE)], p_v)

    lanes = lax.iota(jnp.int32, _RG)
    diag_off = lanes * _NE + lanes
    _NCH = 4

    def group(g, carry):
        flat0 = g * (_RG * _NE)
        base_idx = diag_off + flat0
        best = [jnp.full((_RG,), -1.0, jnp.float32) for _ in range(_NCH)]
        best_i = [jnp.zeros((_RG,), jnp.int32) for _ in range(_NCH)]
        for d in range(_NE):
            c = d % _NCH
            e_vec = jnp.bitwise_and(lanes + d, _NE - 1)
            idx = base_idx + jnp.where(lanes + d >= _NE, d - _NE, d)
            v = plsc.load_gather(p_v, [idx])
            upd = (v > best[c]) | ((v == best[c]) & (e_vec < best_i[c]))
            best[c] = jnp.maximum(v, best[c])
            best_i[c] = jnp.where(upd, e_vec, best_i[c])
        m, mi = best[0], best_i[0]
        for c in range(1, _NCH):
            take = (best[c] > m) | ((best[c] == m) & (best_i[c] < mi))
            m = jnp.where(take, best[c], m)
            mi = jnp.where(take, best_i[c], mi)
        r0 = g * _RG
        w_v[pl.ds(r0, _RG)] = m
        i_v[pl.ds(r0, _RG)] = mi
        return carry

    lax.fori_loop(0, _RW // _RG, group, 0)
    pltpu.sync_copy(w_v, w_hbm.at[pl.ds(base, _RW)])
    pltpu.sync_copy(i_v, i_hbm.at[pl.ds(base, _RW)])


def kernel(x, W, b):
    probs = _tc_softmax(x, W, b)
    wts, idx = _sc_top1(probs.reshape(-1))
    return (wts.reshape(-1, 1), idx.reshape(-1, 1), probs)


# speedup vs baseline: 1.1010x; 1.0103x over previous
import functools

import jax
import jax.numpy as jnp
from jax import lax
from jax.experimental import pallas as pl
from jax.experimental.pallas import tpu as pltpu
from jax.experimental.pallas import tpu_sc as plsc

_DIM = 4096
_NE = 64
_NTOK = 32768
_TB = 1024

_NW = 32
_RW = _NTOK // _NW
_RG = 16


def _softmax_body(x_ref, w_ref, b_ref, probs_ref, p128_ref):
    x = x_ref[...]
    w = w_ref[...]
    logits = lax.dot_general(x, w, (((1,), (1,)), ((), ())))
    logits = logits + b_ref[...]
    m = jnp.max(logits, axis=1, keepdims=True)
    e = jnp.exp(logits - m)
    s = jnp.sum(e, axis=1, keepdims=True)
    probs = e / s
    probs_ref[...] = probs
    p128_ref[...] = jnp.concatenate([probs, jnp.zeros_like(probs)], axis=1)


def _tc_softmax(x, W, b):
    ntok = x.shape[0]
    return pl.pallas_call(
        _softmax_body,
        grid=(ntok // _TB,),
        in_specs=[
            pl.BlockSpec((_TB, _DIM), lambda i: (i, 0)),
            pl.BlockSpec((_NE, _DIM), lambda i: (0, 0)),
            pl.BlockSpec((1, _NE), lambda i: (0, 0)),
        ],
        out_specs=[
            pl.BlockSpec((_TB, _NE), lambda i: (i, 0)),
            pl.BlockSpec((_TB, 2 * _NE), lambda i: (i, 0)),
        ],
        out_shape=[
            jax.ShapeDtypeStruct((ntok, _NE), jnp.float32),
            jax.ShapeDtypeStruct((ntok, 2 * _NE), jnp.float32),
        ],
    )(x, W, b.reshape(1, _NE))


@functools.partial(
    pl.kernel,
    mesh=plsc.VectorSubcoreMesh(core_axis_name="c", subcore_axis_name="s"),
    compiler_params=pltpu.CompilerParams(needs_layout_passes=False),
    out_type=[
        jax.ShapeDtypeStruct((_NTOK,), jnp.float32),
        jax.ShapeDtypeStruct((_NTOK,), jnp.int32),
    ],
    scratch_types=[
        pltpu.VMEM((_RW // 2 * 128,), jnp.float32),
        pltpu.VMEM((_RW,), jnp.float32),
        pltpu.VMEM((_RW,), jnp.int32),
    ],
)
def _sc_top1(p128_hbm, w_hbm, i_hbm, p_v, w_v, i_v):
    wid = lax.axis_index("s") * 2 + lax.axis_index("c")
    base = wid * _RW
    half_rows = _RW // 2

    lanes = lax.iota(jnp.int32, _RG)
    row_off = lanes * 128
    _NCH = 4

    for h in range(2):
        pltpu.sync_copy(
            p128_hbm.at[pl.ds((base + h * half_rows) * 128, half_rows * 128)],
            p_v,
        )

        def group(g, carry):
            base_idx = row_off + g * (_RG * 128)
            best = [jnp.full((_RG,), -1.0, jnp.float32) for _ in range(_NCH)]
            best_i = [jnp.zeros((_RG,), jnp.int32) for _ in range(_NCH)]
            for d in range(_NE):
                c = d % _NCH
                e_vec = jnp.bitwise_and(lanes + d, _NE - 1)
                v = plsc.load_gather(p_v, [base_idx + e_vec])
                upd = (v > best[c]) | ((v == best[c]) & (e_vec < best_i[c]))
                best[c] = jnp.maximum(v, best[c])
                best_i[c] = jnp.where(upd, e_vec, best_i[c])
            m, mi = best[0], best_i[0]
            for c in range(1, _NCH):
                take = (best[c] > m) | ((best[c] == m) & (best_i[c] < mi))
                m = jnp.where(take, best[c], m)
                mi = jnp.where(take, best_i[c], mi)
            r0 = h * half_rows + g * _RG
            w_v[pl.ds(r0, _RG)] = m
            i_v[pl.ds(r0, _RG)] = mi
            return carry

        lax.fori_loop(0, half_rows // _RG, group, 0)

    pltpu.sync_copy(w_v, w_hbm.at[pl.ds(base, _RW)])
    pltpu.sync_copy(i_v, i_hbm.at[pl.ds(base, _RW)])


def kernel(x, W, b):
    probs, p128 = _tc_softmax(x, W, b)
    wts, idx = _sc_top1(p128.reshape(-1))
    return (wts.reshape(-1, 1), idx.reshape(-1, 1), probs)


# speedup vs baseline: 1.1258x; 1.0225x over previous
import functools

import jax
import jax.numpy as jnp
from jax import lax
from jax.experimental import pallas as pl
from jax.experimental.pallas import tpu as pltpu
from jax.experimental.pallas import tpu_sc as plsc

_DIM = 4096
_NE = 64
_NTOK = 32768
_TB = 1024

_NW = 32
_RW = _NTOK // _NW
_RG = 16


def _softmax_body(x_ref, w_ref, b_ref, probs_ref, p128_ref):
    x = x_ref[...]
    w = w_ref[...]
    logits = lax.dot_general(x, w, (((1,), (1,)), ((), ())))
    logits = logits + b_ref[...]
    m = jnp.max(logits, axis=1, keepdims=True)
    e = jnp.exp(logits - m)
    s = jnp.sum(e, axis=1, keepdims=True)
    probs = e / s
    probs_ref[...] = probs
    p128_ref[...] = jnp.concatenate([probs, jnp.zeros_like(probs)], axis=1)


def _tc_softmax(x, W, b):
    ntok = x.shape[0]
    return pl.pallas_call(
        _softmax_body,
        grid=(ntok // _TB,),
        in_specs=[
            pl.BlockSpec((_TB, _DIM), lambda i: (i, 0)),
            pl.BlockSpec((_NE, _DIM), lambda i: (0, 0)),
            pl.BlockSpec((1, _NE), lambda i: (0, 0)),
        ],
        out_specs=[
            pl.BlockSpec((_TB, _NE), lambda i: (i, 0)),
            pl.BlockSpec((_TB, 2 * _NE), lambda i: (i, 0)),
        ],
        out_shape=[
            jax.ShapeDtypeStruct((ntok, _NE), jnp.float32),
            jax.ShapeDtypeStruct((ntok, 2 * _NE), jnp.float32),
        ],
    )(x, W, b.reshape(1, _NE))


@functools.partial(
    pl.kernel,
    mesh=plsc.VectorSubcoreMesh(core_axis_name="c", subcore_axis_name="s"),
    compiler_params=pltpu.CompilerParams(needs_layout_passes=False),
    out_type=[
        jax.ShapeDtypeStruct((_NTOK,), jnp.float32),
        jax.ShapeDtypeStruct((_NTOK,), jnp.int32),
    ],
    scratch_types=[
        pltpu.VMEM((_RW // 2, 128), jnp.float32),
        pltpu.VMEM((_RW,), jnp.float32),
        pltpu.VMEM((_RW,), jnp.int32),
    ],
)
def _sc_top1(p128_hbm, w_hbm, i_hbm, p_v, w_v, i_v):
    wid = lax.axis_index("s") * 2 + lax.axis_index("c")
    base = wid * _RW
    half_rows = _RW // 2

    lanes = lax.iota(jnp.int32, _RG)
    _NCH = 4

    for h in range(2):
        pltpu.sync_copy(
            p128_hbm.at[pl.ds(base + h * half_rows, half_rows), :],
            p_v,
        )

        def group(g, carry):
            rows = g * _RG + lanes
            best = [jnp.full((_RG,), -1.0, jnp.float32) for _ in range(_NCH)]
            best_i = [jnp.zeros((_RG,), jnp.int32) for _ in range(_NCH)]
            for d in range(_NE):
                c = d % _NCH
                e_vec = jnp.bitwise_and(lanes + d, _NE - 1)
                v = plsc.load_gather(p_v, [rows, e_vec])
                upd = (v > best[c]) | ((v == best[c]) & (e_vec < best_i[c]))
                best[c] = jnp.maximum(v, best[c])
                best_i[c] = jnp.where(upd, e_vec, best_i[c])
            m, mi = best[0], best_i[0]
            for c in range(1, _NCH):
                take = (best[c] > m) | ((best[c] == m) & (best_i[c] < mi))
                m = jnp.where(take, best[c], m)
                mi = jnp.where(take, best_i[c], mi)
            r0 = h * half_rows + g * _RG
            w_v[pl.ds(r0, _RG)] = m
            i_v[pl.ds(r0, _RG)] = mi
            return carry

        lax.fori_loop(0, half_rows // _RG, group, 0)

    pltpu.sync_copy(w_v, w_hbm.at[pl.ds(base, _RW)])
    pltpu.sync_copy(i_v, i_hbm.at[pl.ds(base, _RW)])


def kernel(x, W, b):
    probs, p128 = _tc_softmax(x, W, b)
    wts, idx = _sc_top1(p128)
    return (wts.reshape(-1, 1), idx.reshape(-1, 1), probs)
